# baseline (speedup 1.0000x reference)
import functools

import jax
import jax.numpy as jnp
from jax import lax
from jax.experimental import pallas as pl
from jax.experimental.pallas import tpu as pltpu
from jax.experimental.pallas import tpu_sc as plsc

_N = 10000
_E = 320000
_D = 128
_DO = 16
_NC = 2
_NS = 16
_NW = _NC * _NS
_CH = 128
_EPW = 10240
_EPAD = _EPW * _NW
_NCHUNK = _EPW // _CH
_NPAD = 10240
_DUMMY = _N
_RPT = _NPAD // _NS
_BR = 512


def _nb_for(width):
    return 2 if width == _D else 8


def _sc_agg_body(with_counts, width, *refs):
    nb = _nb_for(width)
    it = iter(refs)
    h_hbm, idx_hbm, z_hbm, out_hbm = (next(it) for _ in range(4))
    cnt_hbm = next(it) if with_counts else None
    idxr = next(it)
    rows = [next(it) for _ in range(nb)]
    onesb = next(it) if with_counts else None
    acc = next(it)
    cacc = next(it) if with_counts else None
    gsem = [next(it) for _ in range(nb)]
    ssem = [next(it) for _ in range(nb)]
    csem = [next(it) for _ in range(nb)] if with_counts else None

    c = lax.axis_index("c")
    s = lax.axis_index("s")
    w = s * _NC + c
    nout = _NCHUNK // nb

    for b in range(nb):
        pltpu.sync_copy(idx_hbm.at[w, b], idxr.at[pl.ds(2 * b, 2)])
        pltpu.async_copy(h_hbm.at[idxr.at[2 * b]], rows[b], gsem[b])

    plsc.subcore_barrier()

    def outer(g, _):
        for b in range(nb):
            i = g * nb + b
            pltpu.make_async_copy(h_hbm.at[idxr.at[0]], rows[b], gsem[b]).wait()
            pltpu.async_copy(rows[b], acc.at[idxr.at[2 * b + 1]], ssem[b],
                             add=True)
            if with_counts:
                pltpu.async_copy(onesb, cacc.at[idxr.at[2 * b + 1]], csem[b],
                                 add=True)

            @pl.when(g + 1 < nout)
            def _():
                pltpu.make_async_copy(rows[b], acc.at[idxr.at[1]], ssem[b]).wait()
                if with_counts:
                    pltpu.make_async_copy(onesb, cacc.at[idxr.at[1]], csem[b]).wait()
                pltpu.sync_copy(idx_hbm.at[w, i + nb], idxr.at[pl.ds(2 * b, 2)])
                pltpu.async_copy(h_hbm.at[idxr.at[2 * b]], rows[b], gsem[b])
        return 0
    lax.fori_loop(0, nout, outer, 0)

    for b in range(nb):
        pltpu.make_async_copy(rows[b], acc.at[idxr.at[1]], ssem[b]).wait()
        if with_counts:
            pltpu.make_async_copy(onesb, cacc.at[idxr.at[1]], csem[b]).wait()

    plsc.subcore_barrier()

    pltpu.sync_copy(acc.at[pl.ds(s * _RPT, _RPT)],
                    out_hbm.at[c, pl.ds(s * _RPT, _RPT)])
    if with_counts:
        pltpu.sync_copy(cacc.at[pl.ds(s * _RPT, _RPT)],
                        cnt_hbm.at[c, pl.ds(s * _RPT, _RPT)])


def _sc_agg_full_body(with_counts, width, *refs):
    nb = _nb_for(width)
    it = iter(refs)
    h_hbm, idx_hbm, z_hbm, out_hbm = (next(it) for _ in range(4))
    cnt_hbm = next(it) if with_counts else None
    idxr = next(it)
    rows = [next(it) for _ in range(nb)]
    onesb = next(it) if with_counts else None
    acc = next(it)
    cacc = next(it) if with_counts else None

    s = lax.axis_index("s")

    pltpu.sync_copy(z_hbm, rows[0])

    def zacc(k, _):
        pltpu.sync_copy(rows[0], acc.at[pl.ds(s * _RPT + k * _CH, _CH)])
        if with_counts:
            pltpu.sync_copy(rows[0].at[0], cacc.at[pl.ds(s * _RPT + k * _CH, _CH)])
        return 0
    lax.fori_loop(0, _RPT // _CH, zacc, 0)

    if with_counts:
        def setones(j, _):
            onesb[pl.ds(j * 16, 16)] = jnp.ones((16,), jnp.float32)
            return 0
        lax.fori_loop(0, _CH // 16, setones, 0)

    _sc_agg_body(with_counts, width, *refs)


@functools.cache
def _get_sc_agg(with_counts, width):
    nb = _nb_for(width)
    mesh = plsc.VectorSubcoreMesh(core_axis_name="c", subcore_axis_name="s",
                                  num_cores=_NC, num_subcores=_NS)
    out_type = [jax.ShapeDtypeStruct((_NC, _NPAD, width), jnp.float32)]
    scratch = [pltpu.VMEM((2 * nb, _CH), jnp.int32)]
    scratch += [pltpu.VMEM((_CH, width), jnp.float32) for _ in range(nb)]
    if with_counts:
        out_type.append(jax.ShapeDtypeStruct((_NC, _NPAD), jnp.float32))
        scratch.append(pltpu.VMEM((_CH,), jnp.float32))
    scratch.append(pltpu.VMEM_SHARED((_NPAD, width), jnp.float32))
    if with_counts:
        scratch.append(pltpu.VMEM_SHARED((_NPAD,), jnp.float32))
    scratch += [pltpu.SemaphoreType.DMA for _ in range(2 * nb)]
    if with_counts:
        scratch += [pltpu.SemaphoreType.DMA for _ in range(nb)]
    return pl.kernel(
        functools.partial(_sc_agg_full_body, with_counts, width),
        out_type=out_type,
        mesh=mesh,
        scratch_types=scratch,
        compiler_params=pltpu.CompilerParams(use_tc_tiling_on_sc=False),
    )


def _tc_pre_kernel(h, w, o):
    o[...] = jnp.dot(h[...], w[...], preferred_element_type=jnp.float32)


_tc_pre = pl.pallas_call(
    _tc_pre_kernel,
    grid=(_NPAD // _BR,),
    in_specs=[
        pl.BlockSpec((_BR, _D), lambda i: (i, 0)),
        pl.BlockSpec((_D, _D), lambda i: (0, 0)),
    ],
    out_specs=pl.BlockSpec((_BR, _D), lambda i: (i, 0)),
    out_shape=jax.ShapeDtypeStruct((_NPAD, _D), jnp.float32),
)


def _tc_layer_kernel(last, aggp, inv, r, wl, b, w2l, w2r, o_h, o_z=None, o_r=None):
    a = (aggp[0] + aggp[1]) * inv[...]
    o = jnp.dot(a, wl[...], preferred_element_type=jnp.float32) + b[...] + r[...]
    nrm = jnp.sqrt(jnp.sum(o * o, axis=1, keepdims=True))
    o = o / jnp.maximum(nrm, 1e-12)
    o = jnp.maximum(o, 0.0)
    o_h[...] = o
    if last:
        o_z[...] = jnp.dot(o, w2l[...], preferred_element_type=jnp.float32)
        o_r[...] = jnp.dot(o, w2r[...], preferred_element_type=jnp.float32)


def _make_tc_layer(last):
    grid = (_NPAD // _BR,)
    in_specs = [
        pl.BlockSpec((2, _BR, _D), lambda i: (0, i, 0)),
        pl.BlockSpec((_BR, 1), lambda i: (i, 0)),
        pl.BlockSpec((_BR, _D), lambda i: (i, 0)),
        pl.BlockSpec((_D, _D), lambda i: (0, 0)),
        pl.BlockSpec((1, _D), lambda i: (0, 0)),
        pl.BlockSpec((_D, _DO), lambda i: (0, 0)),
        pl.BlockSpec((_D, _DO), lambda i: (0, 0)),
    ]
    out_shape = [jax.ShapeDtypeStruct((_NPAD, _D), jnp.float32)]
    out_specs = [pl.BlockSpec((_BR, _D), lambda i: (i, 0))]
    if last:
        out_shape += [jax.ShapeDtypeStruct((_NPAD, _DO), jnp.float32)] * 2
        out_specs += [pl.BlockSpec((_BR, _DO), lambda i: (i, 0))] * 2
    return pl.pallas_call(
        functools.partial(_tc_layer_kernel, last),
        grid=grid,
        in_specs=in_specs,
        out_specs=out_specs,
        out_shape=out_shape,
    )


_tc_layer = _make_tc_layer(False)
_tc_layer_last = _make_tc_layer(True)


def _tc_out_kernel(aggzp, inv, r, b2, out):
    o = (aggzp[0] + aggzp[1]) * inv[...] + b2[...] + r[...]
    m = jnp.max(o, axis=1, keepdims=True)
    e = jnp.exp(o - m)
    out[...] = e / jnp.sum(e, axis=1, keepdims=True)


_tc_out = pl.pallas_call(
    _tc_out_kernel,
    grid=(_NPAD // _BR,),
    in_specs=[
        pl.BlockSpec((2, _BR, _DO), lambda i: (0, i, 0)),
        pl.BlockSpec((_BR, 1), lambda i: (i, 0)),
        pl.BlockSpec((_BR, _DO), lambda i: (i, 0)),
        pl.BlockSpec((1, _DO), lambda i: (0, 0)),
    ],
    out_specs=pl.BlockSpec((_BR, _DO), lambda i: (i, 0)),
    out_shape=jax.ShapeDtypeStruct((_NPAD, _DO), jnp.float32),
)


def kernel(x, adj_t, W0_l, b0, W0_r, W1_l, b1, W1_r, W2_l, b2, W2_r):
    src = adj_t[0].astype(jnp.int32)
    dst = adj_t[1].astype(jnp.int32)
    npad_e = _EPAD - _E
    pad_k = jnp.arange(npad_e, dtype=jnp.int32)
    src_p = jnp.concatenate([src, pad_k % _N]).reshape(_NW, _NCHUNK, _CH)
    dst_p = jnp.concatenate(
        [dst, _DUMMY + pad_k % (_NPAD - _N)]).reshape(_NW, _NCHUNK, _CH)
    idx_p = jnp.stack([src_p, dst_p], axis=2)
    h0 = jnp.pad(x, ((0, _NPAD - _N), (0, 0)))

    zeros_d = jnp.zeros((_CH, _D), jnp.float32)
    zeros_o = jnp.zeros((_CH, _DO), jnp.float32)

    agg0p, cntp = _get_sc_agg(True, _D)(h0, idx_p, zeros_d)
    r0 = _tc_pre(h0, W0_r)
    inv = (1.0 / jnp.maximum(cntp[0] + cntp[1], 1.0))[:, None]

    h1 = _tc_layer(agg0p, inv, r0, W0_l, b0[None, :], W2_l, W2_r)[0]
    (agg1p,) = _get_sc_agg(False, _D)(h1, idx_p, zeros_d)
    r1 = _tc_pre(h1, W1_r)
    h2, z, r2 = _tc_layer_last(agg1p, inv, r1, W1_l, b1[None, :], W2_l, W2_r)
    (aggzp,) = _get_sc_agg(False, _DO)(z, idx_p, zeros_o)
    probs = _tc_out(aggzp, inv, r2, b2[None, :])
    return probs[:_N]


# speedup vs baseline: 1.2273x; 1.2273x over previous
import functools

import jax
import jax.numpy as jnp
from jax import lax
from jax.experimental import pallas as pl
from jax.experimental.pallas import tpu as pltpu
from jax.experimental.pallas import tpu_sc as plsc

_N = 10000
_E = 320000
_D = 128
_DO = 16
_NC = 2
_NS = 16
_NW = _NC * _NS
_CH = 128
_EPW = 10240
_EPAD = _EPW * _NW
_NCHUNK = _EPW // _CH
_NPAD = 10240
_DUMMY = _N
_RPT = _NPAD // _NS
_BR = 512


def _nb_for(width):
    return 3 if width == _D else 8


def _ch_for(width):
    return 64 if width == _D else _CH


def _sc_agg_body(with_counts, width, *refs):
    nb = _nb_for(width)
    ch = _ch_for(width)
    nchunk = _EPW // ch
    it = iter(refs)
    h_hbm, idx_hbm, z_hbm, out_hbm = (next(it) for _ in range(4))
    cnt_hbm = next(it) if with_counts else None
    idxr = next(it)
    rows = [next(it) for _ in range(nb)]
    onesb = next(it) if with_counts else None
    acc = next(it)
    cacc = next(it) if with_counts else None
    gsem = [next(it) for _ in range(nb)]
    ssem = [next(it) for _ in range(nb)]
    csem = [next(it) for _ in range(nb)] if with_counts else None

    c = lax.axis_index("c")
    s = lax.axis_index("s")
    w = s * _NC + c
    nout = nchunk // nb

    pltpu.sync_copy(idx_hbm.at[w], idxr)
    for b in range(nb):
        pltpu.async_copy(h_hbm.at[idxr.at[2 * b]], rows[b], gsem[b])

    plsc.subcore_barrier()

    def outer(g, _):
        for b in range(nb):
            i = g * nb + b
            pltpu.make_async_copy(h_hbm.at[idxr.at[0]], rows[b], gsem[b]).wait()
            pltpu.async_copy(rows[b], acc.at[idxr.at[2 * i + 1]], ssem[b],
                             add=True)
            if with_counts:
                pltpu.async_copy(onesb, cacc.at[idxr.at[2 * i + 1]], csem[b],
                                 add=True)

            @pl.when(g + 1 < nout)
            def _():
                pltpu.make_async_copy(rows[b], acc.at[idxr.at[1]], ssem[b]).wait()
                if with_counts:
                    pltpu.make_async_copy(onesb, cacc.at[idxr.at[1]], csem[b]).wait()
                pltpu.async_copy(h_hbm.at[idxr.at[2 * (i + nb)]], rows[b], gsem[b])
        return 0
    lax.fori_loop(0, nout, outer, 0)

    for b in range(nb):
        pltpu.make_async_copy(rows[b], acc.at[idxr.at[1]], ssem[b]).wait()
        if with_counts:
            pltpu.make_async_copy(onesb, cacc.at[idxr.at[1]], csem[b]).wait()

    plsc.subcore_barrier()

    pltpu.sync_copy(acc.at[pl.ds(s * _RPT, _RPT)],
                    out_hbm.at[c, pl.ds(s * _RPT, _RPT)])
    if with_counts:
        pltpu.sync_copy(cacc.at[pl.ds(s * _RPT, _RPT)],
                        cnt_hbm.at[c, pl.ds(s * _RPT, _RPT)])


def _sc_agg_full_body(with_counts, width, *refs):
    nb = _nb_for(width)
    it = iter(refs)
    h_hbm, idx_hbm, z_hbm, out_hbm = (next(it) for _ in range(4))
    cnt_hbm = next(it) if with_counts else None
    idxr = next(it)
    rows = [next(it) for _ in range(nb)]
    onesb = next(it) if with_counts else None
    acc = next(it)
    cacc = next(it) if with_counts else None

    s = lax.axis_index("s")
    ch = _ch_for(width)

    pltpu.sync_copy(z_hbm, rows[0])

    def zacc(k, _):
        pltpu.sync_copy(rows[0], acc.at[pl.ds(s * _RPT + k * ch, ch)])
        return 0
    lax.fori_loop(0, _RPT // ch, zacc, 0)

    if with_counts:
        def czacc(k, _):
            pltpu.sync_copy(rows[0].at[0],
                            cacc.at[pl.ds(s * _RPT + k * width, width)])
            return 0
        lax.fori_loop(0, _RPT // width, czacc, 0)

        def setones(j, _):
            onesb[pl.ds(j * 16, 16)] = jnp.ones((16,), jnp.float32)
            return 0
        lax.fori_loop(0, ch // 16, setones, 0)

    _sc_agg_body(with_counts, width, *refs)


@functools.cache
def _get_sc_agg(with_counts, width):
    nb = _nb_for(width)
    ch = _ch_for(width)
    nchunk = _EPW // ch
    mesh = plsc.VectorSubcoreMesh(core_axis_name="c", subcore_axis_name="s",
                                  num_cores=_NC, num_subcores=_NS)
    out_type = [jax.ShapeDtypeStruct((_NC, _NPAD, width), jnp.float32)]
    scratch = [pltpu.VMEM((2 * nchunk, ch), jnp.int32)]
    scratch += [pltpu.VMEM((ch, width), jnp.float32) for _ in range(nb)]
    if with_counts:
        out_type.append(jax.ShapeDtypeStruct((_NC, _NPAD), jnp.float32))
        scratch.append(pltpu.VMEM((ch,), jnp.float32))
    scratch.append(pltpu.VMEM_SHARED((_NPAD, width), jnp.float32))
    if with_counts:
        scratch.append(pltpu.VMEM_SHARED((_NPAD,), jnp.float32))
    scratch += [pltpu.SemaphoreType.DMA for _ in range(2 * nb)]
    if with_counts:
        scratch += [pltpu.SemaphoreType.DMA for _ in range(nb)]
    return pl.kernel(
        functools.partial(_sc_agg_full_body, with_counts, width),
        out_type=out_type,
        mesh=mesh,
        scratch_types=scratch,
        compiler_params=pltpu.CompilerParams(use_tc_tiling_on_sc=False),
    )


def _tc_pre_kernel(h, w, o):
    o[...] = jnp.dot(h[...], w[...], preferred_element_type=jnp.float32)


_tc_pre = pl.pallas_call(
    _tc_pre_kernel,
    grid=(_NPAD // _BR,),
    in_specs=[
        pl.BlockSpec((_BR, _D), lambda i: (i, 0)),
        pl.BlockSpec((_D, _D), lambda i: (0, 0)),
    ],
    out_specs=pl.BlockSpec((_BR, _D), lambda i: (i, 0)),
    out_shape=jax.ShapeDtypeStruct((_NPAD, _D), jnp.float32),
)


def _tc_layer_kernel(last, aggp, inv, r, wl, b, w2l, w2r, o_h, o_z=None, o_r=None):
    a = (aggp[0] + aggp[1]) * inv[...]
    o = jnp.dot(a, wl[...], preferred_element_type=jnp.float32) + b[...] + r[...]
    nrm = jnp.sqrt(jnp.sum(o * o, axis=1, keepdims=True))
    o = o / jnp.maximum(nrm, 1e-12)
    o = jnp.maximum(o, 0.0)
    o_h[...] = o
    if last:
        o_z[...] = jnp.dot(o, w2l[...], preferred_element_type=jnp.float32)
        o_r[...] = jnp.dot(o, w2r[...], preferred_element_type=jnp.float32)


def _make_tc_layer(last):
    grid = (_NPAD // _BR,)
    in_specs = [
        pl.BlockSpec((2, _BR, _D), lambda i: (0, i, 0)),
        pl.BlockSpec((_BR, 1), lambda i: (i, 0)),
        pl.BlockSpec((_BR, _D), lambda i: (i, 0)),
        pl.BlockSpec((_D, _D), lambda i: (0, 0)),
        pl.BlockSpec((1, _D), lambda i: (0, 0)),
        pl.BlockSpec((_D, _DO), lambda i: (0, 0)),
        pl.BlockSpec((_D, _DO), lambda i: (0, 0)),
    ]
    out_shape = [jax.ShapeDtypeStruct((_NPAD, _D), jnp.float32)]
    out_specs = [pl.BlockSpec((_BR, _D), lambda i: (i, 0))]
    if last:
        out_shape += [jax.ShapeDtypeStruct((_NPAD, _DO), jnp.float32)] * 2
        out_specs += [pl.BlockSpec((_BR, _DO), lambda i: (i, 0))] * 2
    return pl.pallas_call(
        functools.partial(_tc_layer_kernel, last),
        grid=grid,
        in_specs=in_specs,
        out_specs=out_specs,
        out_shape=out_shape,
    )


_tc_layer = _make_tc_layer(False)
_tc_layer_last = _make_tc_layer(True)


def _tc_out_kernel(aggzp, inv, r, b2, out):
    o = (aggzp[0] + aggzp[1]) * inv[...] + b2[...] + r[...]
    m = jnp.max(o, axis=1, keepdims=True)
    e = jnp.exp(o - m)
    out[...] = e / jnp.sum(e, axis=1, keepdims=True)


_tc_out = pl.pallas_call(
    _tc_out_kernel,
    grid=(_NPAD // _BR,),
    in_specs=[
        pl.BlockSpec((2, _BR, _DO), lambda i: (0, i, 0)),
        pl.BlockSpec((_BR, 1), lambda i: (i, 0)),
        pl.BlockSpec((_BR, _DO), lambda i: (i, 0)),
        pl.BlockSpec((1, _DO), lambda i: (0, 0)),
    ],
    out_specs=pl.BlockSpec((_BR, _DO), lambda i: (i, 0)),
    out_shape=jax.ShapeDtypeStruct((_NPAD, _DO), jnp.float32),
)


def kernel(x, adj_t, W0_l, b0, W0_r, W1_l, b1, W1_r, W2_l, b2, W2_r):
    src = adj_t[0].astype(jnp.int32)
    dst = adj_t[1].astype(jnp.int32)
    npad_e = _EPAD - _E
    pad_k = jnp.arange(npad_e, dtype=jnp.int32)
    src_flat = jnp.concatenate([src, pad_k % _N])
    dst_flat = jnp.concatenate([dst, _DUMMY + pad_k % (_NPAD - _N)])

    def pack_idx(ch):
        nchunk = _EPW // ch
        sp = src_flat.reshape(_NW, nchunk, ch)
        dp = dst_flat.reshape(_NW, nchunk, ch)
        return jnp.stack([sp, dp], axis=2).reshape(_NW, 2 * nchunk, ch)

    idx_d = pack_idx(_ch_for(_D))
    idx_o = pack_idx(_ch_for(_DO))
    h0 = jnp.pad(x, ((0, _NPAD - _N), (0, 0)))

    zeros_d = jnp.zeros((_ch_for(_D), _D), jnp.float32)
    zeros_o = jnp.zeros((_ch_for(_DO), _DO), jnp.float32)

    agg0p, cntp = _get_sc_agg(True, _D)(h0, idx_d, zeros_d)
    r0 = _tc_pre(h0, W0_r)
    inv = (1.0 / jnp.maximum(cntp[0] + cntp[1], 1.0))[:, None]

    h1 = _tc_layer(agg0p, inv, r0, W0_l, b0[None, :], W2_l, W2_r)[0]
    (agg1p,) = _get_sc_agg(False, _D)(h1, idx_d, zeros_d)
    r1 = _tc_pre(h1, W1_r)
    h2, z, r2 = _tc_layer_last(agg1p, inv, r1, W1_l, b1[None, :], W2_l, W2_r)
    (aggzp,) = _get_sc_agg(False, _DO)(z, idx_o, zeros_o)
    probs = _tc_out(aggzp, inv, r2, b2[None, :])
    return probs[:_N]
